# baseline (speedup 1.0000x reference)
import jax
import jax.numpy as jnp
from jax.experimental import pallas as pl
from jax.experimental.pallas import tpu as pltpu

_ALPHA = 0.1
_IGNORE = 0


def _softmax_parts(xt_ref, c_ref):
    x0 = xt_ref[0].astype(jnp.float32)
    x1 = xt_ref[1].astype(jnp.float32)
    x2 = xt_ref[2].astype(jnp.float32)
    x3 = xt_ref[3].astype(jnp.float32)
    c = c_ref[...].astype(jnp.int32)
    t = jnp.bitwise_and(c, 3)

    m = jnp.maximum(jnp.maximum(x0, x1), jnp.maximum(x2, x3))
    e0 = jnp.exp(x0 - m)
    e1 = jnp.exp(x1 - m)
    e2 = jnp.exp(x2 - m)
    e3 = jnp.exp(x3 - m)
    z = (e0 + e1) + (e2 + e3)
    r = 1.0 / z
    logz = jnp.log(z)

    t0 = t == 0
    t1 = t == 1
    t2 = t == 2
    x_t = jnp.where(t0, x0, jnp.where(t1, x1, jnp.where(t2, x2, x3)))
    e_t = jnp.where(t0, e0, jnp.where(t1, e1, jnp.where(t2, e2, e3)))
    mask = jnp.where(t0, 0.0, 1.0)

    nll_sum = jnp.sum((m + logz - x_t) * mask)
    cnt = jnp.sum(mask)
    return c, (e0, e1, e2, e3), e_t, r, nll_sum, cnt


def _pack_out(out_ref, nll_sum, cnt, k1_sum, cross_sum):
    lane = jax.lax.broadcasted_iota(jnp.int32, (1, 1, 128), 2)
    out_ref[...] = jnp.where(
        lane == 0, nll_sum,
        jnp.where(lane == 1, cnt,
                  jnp.where(lane == 2, k1_sum,
                            jnp.where(lane == 3, cross_sum, 0.0))))


def _body_general(xt_ref, c_ref, w_ref, k1_ref, out_ref):
    c, (e0, e1, e2, e3), e_t, r, nll_sum, cnt = _softmax_parts(xt_ref, c_ref)
    wacc = jnp.zeros_like(r)
    k1acc = jnp.zeros_like(r)
    for cc in range(16):
        sel = c == cc
        dot = (w_ref[cc, 0] * e0 + w_ref[cc, 1] * e1
               + w_ref[cc, 2] * e2 + w_ref[cc, 3] * e3)
        wacc += jnp.where(sel, dot, 0.0)
        k1acc += jnp.where(sel, k1_ref[cc], 0.0)
    _pack_out(out_ref, nll_sum, cnt, jnp.sum(k1acc), jnp.sum(wacc * r))


def _body_fast(xt_ref, c_ref, p_ref, out_ref):
    _, _, e_t, r, nll_sum, cnt = _softmax_parts(xt_ref, c_ref)
    nt = float(e_t.shape[0] * e_t.shape[1])
    pt_sum = jnp.sum(e_t * r)
    k1_sum = p_ref[2] * nt
    cross_sum = p_ref[0] * nt + p_ref[1] * pt_sum
    _pack_out(out_ref, nll_sum, cnt, k1_sum, cross_sum)


def kernel(input, target, dummy, labels, matric):
    B, T, V = input.shape
    N = B * T
    length = labels.shape[1] + 1.0
    s = 1.0 - (1.0 - _ALPHA) ** (1.0 / length)

    M = matric[:-1, :-1, :-1]

    wfull = s * M.reshape(16, V)
    tidx = jnp.arange(16) % 4
    w16 = wfull.at[jnp.arange(16), tidx].set(1.0 - jnp.sum(wfull, axis=1))
    k1 = jnp.sum(jax.scipy.special.xlogy(w16, w16), axis=1)

    u = M[0, 0, 0]
    su = s * u
    k1c = (3.0 * jax.scipy.special.xlogy(su, su)
           + jax.scipy.special.xlogy(1.0 - 4.0 * su, 1.0 - 4.0 * su))
    params = jnp.stack([su, 1.0 - 5.0 * su, k1c, jnp.float32(0.0)])
    uniform = jnp.all(M == u)

    xt = jnp.moveaxis(input, 2, 0).astype(jnp.bfloat16)

    bblk = 1024
    grid = B // bblk

    def call(body, codes, *tables):
        n_smem = len(tables)
        return pl.pallas_call(
            body,
            grid=(grid,),
            in_specs=[
                pl.BlockSpec((V, bblk, T), lambda i: (0, i, 0)),
                pl.BlockSpec((bblk, T), lambda i: (i, 0)),
            ] + [pl.BlockSpec(memory_space=pltpu.SMEM)] * n_smem,
            out_specs=pl.BlockSpec((1, 1, 128), lambda i: (i, 0, 0)),
            out_shape=jax.ShapeDtypeStruct((grid, 1, 128), jnp.float32),
            compiler_params=pltpu.CompilerParams(
                dimension_semantics=("parallel",)),
        )(xt, codes, *tables)

    def general_path():
        forth = jnp.concatenate(
            [jnp.zeros((B, 1), target.dtype), target[:, :-1]], axis=1)
        c = (forth * 4 + target).astype(jnp.int32)
        return call(_body_general, c, w16, k1)

    out = jax.lax.cond(
        uniform,
        lambda: call(_body_fast, target, params),
        general_path)

    parts = jnp.sum(out, axis=(0, 1))
    ce = parts[0] / parts[1]
    kl = (parts[2] - parts[3]) / (N * V)
    return ce + kl


# speedup vs baseline: 1.0199x; 1.0199x over previous
import jax
import jax.numpy as jnp
from jax.experimental import pallas as pl
from jax.experimental.pallas import tpu as pltpu

_ALPHA = 0.1
_IGNORE = 0


def _softmax_parts(xt_ref, c_ref):
    x0 = xt_ref[0].astype(jnp.float32)
    x1 = xt_ref[1].astype(jnp.float32)
    x2 = xt_ref[2].astype(jnp.float32)
    x3 = xt_ref[3].astype(jnp.float32)
    c = c_ref[...].astype(jnp.int32)
    t = jnp.bitwise_and(c, 3)

    m = jnp.maximum(jnp.maximum(x0, x1), jnp.maximum(x2, x3))
    e0 = jnp.exp(x0 - m)
    e1 = jnp.exp(x1 - m)
    e2 = jnp.exp(x2 - m)
    e3 = jnp.exp(x3 - m)
    z = (e0 + e1) + (e2 + e3)
    r = 1.0 / z
    logz = jnp.log(z)

    t0 = t == 0
    t1 = t == 1
    t2 = t == 2
    x_t = jnp.where(t0, x0, jnp.where(t1, x1, jnp.where(t2, x2, x3)))
    e_t = jnp.where(t0, e0, jnp.where(t1, e1, jnp.where(t2, e2, e3)))
    mask = jnp.where(t0, 0.0, 1.0)

    nll_sum = jnp.sum((m + logz - x_t) * mask)
    cnt = jnp.sum(mask)
    return c, (e0, e1, e2, e3), e_t, r, nll_sum, cnt


def _pack_out(out_ref, nll_sum, cnt, k1_sum, cross_sum):
    lane = jax.lax.broadcasted_iota(jnp.int32, (1, 1, 128), 2)
    out_ref[...] = jnp.where(
        lane == 0, nll_sum,
        jnp.where(lane == 1, cnt,
                  jnp.where(lane == 2, k1_sum,
                            jnp.where(lane == 3, cross_sum, 0.0))))


def _body_general(xt_ref, c_ref, w_ref, k1_ref, out_ref):
    c, (e0, e1, e2, e3), e_t, r, nll_sum, cnt = _softmax_parts(xt_ref, c_ref)
    wacc = jnp.zeros_like(r)
    k1acc = jnp.zeros_like(r)
    for cc in range(16):
        sel = c == cc
        dot = (w_ref[cc, 0] * e0 + w_ref[cc, 1] * e1
               + w_ref[cc, 2] * e2 + w_ref[cc, 3] * e3)
        wacc += jnp.where(sel, dot, 0.0)
        k1acc += jnp.where(sel, k1_ref[cc], 0.0)
    _pack_out(out_ref, nll_sum, cnt, jnp.sum(k1acc), jnp.sum(wacc * r))


def _body_fast(xt_ref, c_ref, p_ref, out_ref):
    _, _, e_t, r, nll_sum, cnt = _softmax_parts(xt_ref, c_ref)
    nt = float(e_t.shape[0] * e_t.shape[1])
    pt_sum = jnp.sum(e_t * r)
    k1_sum = p_ref[2] * nt
    cross_sum = p_ref[0] * nt + p_ref[1] * pt_sum
    _pack_out(out_ref, nll_sum, cnt, k1_sum, cross_sum)


def kernel(input, target, dummy, labels, matric):
    B, T, V = input.shape
    N = B * T
    length = labels.shape[1] + 1.0
    s = 1.0 - (1.0 - _ALPHA) ** (1.0 / length)

    M = matric[:-1, :-1, :-1]

    wfull = s * M.reshape(16, V)
    tidx = jnp.arange(16) % 4
    w16 = wfull.at[jnp.arange(16), tidx].set(1.0 - jnp.sum(wfull, axis=1))
    k1 = jnp.sum(jax.scipy.special.xlogy(w16, w16), axis=1)

    u = M[0, 0, 0]
    su = s * u
    k1c = (3.0 * jax.scipy.special.xlogy(su, su)
           + jax.scipy.special.xlogy(1.0 - 4.0 * su, 1.0 - 4.0 * su))
    params = jnp.stack([su, 1.0 - 5.0 * su, k1c, jnp.float32(0.0)])
    uniform = jnp.all(M == u)

    xt = jnp.moveaxis(input, 2, 0).astype(jnp.bfloat16)

    bblk = 512
    grid = B // bblk

    def call(body, codes, *tables):
        n_smem = len(tables)
        return pl.pallas_call(
            body,
            grid=(grid,),
            in_specs=[
                pl.BlockSpec((V, bblk, T), lambda i: (0, i, 0)),
                pl.BlockSpec((bblk, T), lambda i: (i, 0)),
            ] + [pl.BlockSpec(memory_space=pltpu.SMEM)] * n_smem,
            out_specs=pl.BlockSpec((1, 1, 128), lambda i: (i, 0, 0)),
            out_shape=jax.ShapeDtypeStruct((grid, 1, 128), jnp.float32),
            compiler_params=pltpu.CompilerParams(
                dimension_semantics=("parallel",)),
        )(xt, codes, *tables)

    def general_path():
        forth = jnp.concatenate(
            [jnp.zeros((B, 1), target.dtype), target[:, :-1]], axis=1)
        c = (forth * 4 + target).astype(jnp.int32)
        return call(_body_general, c, w16, k1)

    out = jax.lax.cond(
        uniform,
        lambda: call(_body_fast, target, params),
        general_path)

    parts = jnp.sum(out, axis=(0, 1))
    ce = parts[0] / parts[1]
    kl = (parts[2] - parts[3]) / (N * V)
    return ce + kl
